# baseline (speedup 1.0000x reference)
import jax
import jax.numpy as jnp
import numpy as np
from jax import lax
from jax.experimental import pallas as pl
from jax.experimental.pallas import tpu as pltpu
from jax.experimental.pallas import tpu_sc as plsc

FIELD = 26
K = 8
D = FIELD * K
BATCH = 4096
NW = 32
SPW = BATCH // NW
CH = 4
ROWS = CH * FIELD
NCH = SPW // CH
PAIRS = [(i, j) for i in range(FIELD) for j in range(i + 1, FIELD)]
PV = (len(PAIRS) * K + 15) // 16


def _pair_tables():
    n = PV * 16
    a = np.zeros(n, np.int32)
    b = np.zeros(n, np.int32)
    for t in range(len(PAIRS) * K):
        p, k = divmod(t, K)
        i, j = PAIRS[p]
        a[t] = (i << 8) | (j * K + k)
        b[t] = (j << 8) | (i * K + k)
    return a.reshape(PV, 16), b.reshape(PV, 16)


def _w_tables():
    t = np.zeros((2 * CH, 16), np.int32)
    for s in range(CH):
        for l in range(16):
            t[2 * s, l] = s * FIELD + l
            t[2 * s + 1, l] = s * FIELD + 16 + l if l < 10 else s * FIELD
    return t


_A_TAB, _B_TAB = _pair_tables()
_W_TAB = _w_tables()


def _ffm_body(idx_hbm, v_hbm, w_hbm, w0_hbm, atab_hbm, btab_hbm, wtab_hbm,
              out_hbm, idx_v, vblk0, vblk1, wblk0, wblk1, atab_v, btab_v,
              wtab_v, w0_v, acc_v, out_v, sv0, sv1, sw0, sw1):
    wid = lax.axis_index("s") * 2 + lax.axis_index("c")
    lane = lax.iota(jnp.int32, 16)
    lane8 = lane < 8
    lane10 = lane < 10
    zero16 = jnp.zeros((16,), jnp.int32)

    pltpu.sync_copy(idx_hbm.at[pl.ds(wid * NCH, NCH)], idx_v)
    pltpu.sync_copy(atab_hbm, atab_v)
    pltpu.sync_copy(btab_hbm, btab_v)
    pltpu.sync_copy(wtab_hbm, wtab_v)
    pltpu.sync_copy(w0_hbm, w0_v)

    def issue(c, vbuf, wbuf, semv, semw):
        pltpu.async_copy(v_hbm.at[idx_v.at[c]], vbuf, semv)
        pltpu.async_copy(w_hbm.at[idx_v.at[c]], wbuf, semw)

    def wait(c, vbuf, wbuf, semv, semw):
        pltpu.make_async_copy(v_hbm.at[idx_v.at[c]], vbuf, semv).wait()
        pltpu.make_async_copy(w_hbm.at[idx_v.at[c]], wbuf, semw).wait()

    def compute(c, vbuf, wbuf):
        accs = []
        for s in range(CH):
            g1 = plsc.load_gather(wbuf, [wtab_v[2 * s, :]])
            g2 = plsc.load_gather(wbuf, [wtab_v[2 * s + 1, :]])
            accs.append(g1 + jnp.where(lane10, g2, 0.0))

        def pv_body(pv, accs):
            pa = atab_v[pv, :]
            pb = btab_v[pv, :]
            ar, ac = pa >> 8, pa & 255
            br, bc = pb >> 8, pb & 255
            out = []
            for s in range(CH):
                av = plsc.load_gather(vbuf, [ar + s * FIELD, ac])
                bv = plsc.load_gather(vbuf, [br + s * FIELD, bc])
                out.append(accs[s] + av * bv)
            return tuple(out)

        accs = lax.fori_loop(0, PV - 1, pv_body, tuple(accs))

        pa = atab_v[PV - 1, :]
        pb = btab_v[PV - 1, :]
        ar, ac = pa >> 8, pa & 255
        br, bc = pb >> 8, pb & 255
        for s in range(CH):
            av = plsc.load_gather(vbuf, [ar + s * FIELD, ac])
            bv = plsc.load_gather(vbuf, [br + s * FIELD, bc])
            acc_v[c * CH + s, :] = accs[s] + jnp.where(lane8, av * bv, 0.0)

    issue(0, vblk0, wblk0, sv0, sw0)

    def step(i, carry):
        c0 = 2 * i
        wait(c0, vblk0, wblk0, sv0, sw0)
        issue(c0 + 1, vblk1, wblk1, sv1, sw1)
        compute(c0, vblk0, wblk0)
        wait(c0 + 1, vblk1, wblk1, sv1, sw1)

        @pl.when(c0 + 2 < NCH)
        def _():
            issue(c0 + 2, vblk0, wblk0, sv0, sw0)

        compute(c0 + 1, vblk1, wblk1)
        return carry

    lax.fori_loop(0, NCH // 2, step, 0)

    w0s = w0_v[...]
    for g in range(SPW // 16):
        rows = g * 16 + lane
        tot = jnp.zeros((16,), jnp.float32)
        for j in range(16):
            tot = tot + plsc.load_gather(
                acc_v, [rows, jnp.full((16,), j, jnp.int32)])
        x = tot + w0s
        out_v[pl.ds(g * 16, 16)] = 1.0 / (1.0 + jnp.exp(-x))
    pltpu.sync_copy(out_v, out_hbm.at[pl.ds(wid * SPW, SPW)])


@jax.jit
def kernel(inputs, w0, w, v):
    feat_num = v.shape[0] // FIELD
    offs = (jnp.arange(FIELD, dtype=jnp.int32) * feat_num)[None, :]
    idx = (inputs + offs).reshape(BATCH * FIELD // ROWS, ROWS)
    v2 = v.reshape(v.shape[0], D)
    wf = w.reshape(-1)
    w0p = jnp.broadcast_to(w0, (16,))

    mesh = plsc.VectorSubcoreMesh(core_axis_name="c", subcore_axis_name="s")
    run = pl.kernel(
        _ffm_body,
        out_type=jax.ShapeDtypeStruct((BATCH,), jnp.float32),
        mesh=mesh,
        compiler_params=pltpu.CompilerParams(
            use_tc_tiling_on_sc=False, needs_layout_passes=False),
        scratch_types=[
            pltpu.VMEM((NCH, ROWS), jnp.int32),
            pltpu.VMEM((ROWS, D), jnp.float32),
            pltpu.VMEM((ROWS, D), jnp.float32),
            pltpu.VMEM((ROWS,), jnp.float32),
            pltpu.VMEM((ROWS,), jnp.float32),
            pltpu.VMEM((PV, 16), jnp.int32),
            pltpu.VMEM((PV, 16), jnp.int32),
            pltpu.VMEM((2 * CH, 16), jnp.int32),
            pltpu.VMEM((16,), jnp.float32),
            pltpu.VMEM((SPW, 16), jnp.float32),
            pltpu.VMEM((SPW,), jnp.float32),
            pltpu.SemaphoreType.DMA,
            pltpu.SemaphoreType.DMA,
            pltpu.SemaphoreType.DMA,
            pltpu.SemaphoreType.DMA,
        ],
    )
    return run(idx, v2, wf, w0p, jnp.asarray(_A_TAB), jnp.asarray(_B_TAB),
               jnp.asarray(_W_TAB))
